# baseline (speedup 1.0000x reference)
import jax
import jax.numpy as jnp
from jax import lax
from jax.experimental import pallas as pl
from jax.experimental.pallas import tpu as pltpu
from jax.experimental.pallas import tpu_sc as plsc

CH = 32
NBUF = 8


def _sc_time_gather(ti_hbm, wtime_hbm, tm_hbm, idx_v, tm_v, sem):
    nc = 2
    wid = lax.axis_index("s") * nc + lax.axis_index("c")

    @pl.when(wid == 0)
    def _():
        pltpu.sync_copy(ti_hbm, idx_v)
        pltpu.async_copy(wtime_hbm.at[idx_v], tm_v, sem).wait()
        pltpu.sync_copy(tm_v, tm_hbm)


def _tc_ring(ti_ref, wft_ref, wtk_ref, tm_ref, x_ref, o_ref,
             in_buf, out_buf, ef_ref, eb_ref, in_sem, out_sem):
    num_features, batch, d_model = x_ref.shape
    feature_types, third = wft_ref.shape
    n_tickers = wtk_ref.shape[0]
    n_chunks = num_features // CH

    def in_copy(c, b):
        return pltpu.make_async_copy(
            x_ref.at[pl.ds(c * CH, CH)], in_buf.at[b], in_sem.at[b])

    def out_copy(c, b):
        return pltpu.make_async_copy(
            out_buf.at[b], o_ref.at[pl.ds(c * CH, CH)], out_sem.at[b])

    for b in range(NBUF):
        in_copy(b, b).start()

    f = lax.broadcasted_iota(jnp.int32, (num_features, 1), 0)
    g = lax.broadcasted_iota(jnp.int32, (1, feature_types), 1)
    t = lax.broadcasted_iota(jnp.int32, (1, n_tickers), 1)
    sel_ft = (f // n_tickers == g).astype(jnp.float32)
    sel_tk = (f % n_tickers == t).astype(jnp.float32)
    ftrows = jnp.dot(sel_ft, wft_ref[...], preferred_element_type=jnp.float32)
    tkrows = jnp.dot(sel_tk, wtk_ref[...], preferred_element_type=jnp.float32)

    col = lax.broadcasted_iota(jnp.int32, (third, d_model), 1)
    row = lax.broadcasted_iota(jnp.int32, (third, d_model), 0)
    base = col - 3 * row
    p0 = (base == 0).astype(jnp.float32)
    p1 = (base == 1).astype(jnp.float32)
    p2 = (base == 2).astype(jnp.float32)
    ef_ref[...] = (jnp.dot(ftrows, p0, preferred_element_type=jnp.float32)
                   + jnp.dot(tkrows, p2, preferred_element_type=jnp.float32))
    eb_ref[...] = jnp.dot(tm_ref[...], p1, preferred_element_type=jnp.float32)

    def body(i, _):
        b = lax.rem(i, NBUF)
        in_copy(i, b).wait()

        @pl.when(i >= NBUF)
        def _():
            out_copy(i - NBUF, b).wait()

        ef = ef_ref[pl.ds(i * CH, CH), :]
        out_buf[b] = in_buf[b] + ef[:, None, :] + eb_ref[...][None, :, :]
        out_copy(i, b).start()

        @pl.when(i + NBUF < n_chunks)
        def _():
            in_copy(i + NBUF, b).start()

        return 0

    lax.fori_loop(0, n_chunks, body, 0)

    for b in range(NBUF):
        out_copy(0, b).wait()


@jax.jit
def kernel(x, time_indices, W_ft, W_time, W_tk):
    num_features, batch, d_model = x.shape
    third = W_ft.shape[1]
    ti = time_indices.astype(jnp.int32)

    mesh = plsc.VectorSubcoreMesh(core_axis_name="c", subcore_axis_name="s",
                                  num_cores=2, num_subcores=16)
    tm = pl.kernel(
        _sc_time_gather,
        out_type=jax.ShapeDtypeStruct((batch, third), jnp.float32),
        mesh=mesh,
        scratch_types=[
            pltpu.VMEM((batch,), jnp.int32),
            pltpu.VMEM((batch, third), jnp.float32),
            pltpu.SemaphoreType.DMA,
        ],
    )(ti, W_time)

    return pl.pallas_call(
        _tc_ring,
        in_specs=[
            pl.BlockSpec(memory_space=pltpu.MemorySpace.VMEM),
            pl.BlockSpec(memory_space=pltpu.MemorySpace.VMEM),
            pl.BlockSpec(memory_space=pltpu.MemorySpace.VMEM),
            pl.BlockSpec(memory_space=pltpu.MemorySpace.VMEM),
            pl.BlockSpec(memory_space=pl.ANY),
        ],
        out_specs=pl.BlockSpec(memory_space=pl.ANY),
        out_shape=jax.ShapeDtypeStruct(x.shape, x.dtype),
        scratch_shapes=[
            pltpu.VMEM((NBUF, CH, batch, d_model), jnp.float32),
            pltpu.VMEM((NBUF, CH, batch, d_model), jnp.float32),
            pltpu.VMEM((num_features, d_model), jnp.float32),
            pltpu.VMEM((batch, d_model), jnp.float32),
            pltpu.SemaphoreType.DMA((NBUF,)),
            pltpu.SemaphoreType.DMA((NBUF,)),
        ],
    )(ti.reshape(batch, 1), W_ft, W_tk, tm, x)


# speedup vs baseline: 1.3083x; 1.3083x over previous
import jax
import jax.numpy as jnp
from jax import lax
from jax.experimental import pallas as pl
from jax.experimental.pallas import tpu as pltpu

CH = 64
NBUF = 4


def _pipeline_kernel(ti_ref, wft_ref, wtime_ref, wtk_ref, x_ref, o_ref,
                     in_buf, out_buf, ef_ref, eb_ref, in_sem, out_sem):
    num_features, batch, d_model = x_ref.shape
    feature_types, third = wft_ref.shape
    n_tickers = wtk_ref.shape[0]
    max_time = wtime_ref.shape[0]
    n_chunks = num_features // CH

    def in_copy(c, b):
        return pltpu.make_async_copy(
            x_ref.at[pl.ds(c * CH, CH)], in_buf.at[b], in_sem.at[b])

    def out_copy(c, b):
        return pltpu.make_async_copy(
            out_buf.at[b], o_ref.at[pl.ds(c * CH, CH)], out_sem.at[b])

    for b in range(NBUF):
        in_copy(b, b).start()

    f = lax.broadcasted_iota(jnp.int32, (num_features, 1), 0)
    g = lax.broadcasted_iota(jnp.int32, (1, feature_types), 1)
    t = lax.broadcasted_iota(jnp.int32, (1, n_tickers), 1)
    sel_ft = (f // n_tickers == g).astype(jnp.float32)
    sel_tk = (f % n_tickers == t).astype(jnp.float32)
    ftrows = jnp.dot(sel_ft, wft_ref[...], preferred_element_type=jnp.float32)
    tkrows = jnp.dot(sel_tk, wtk_ref[...], preferred_element_type=jnp.float32)

    col = lax.broadcasted_iota(jnp.int32, (third, d_model), 1)
    row = lax.broadcasted_iota(jnp.int32, (third, d_model), 0)
    base = col - 3 * row
    p0 = (base == 0).astype(jnp.float32)
    p1 = (base == 1).astype(jnp.float32)
    p2 = (base == 2).astype(jnp.float32)
    ef_ref[...] = (jnp.dot(ftrows, p0, preferred_element_type=jnp.float32)
                   + jnp.dot(tkrows, p2, preferred_element_type=jnp.float32))

    t_iota = lax.broadcasted_iota(jnp.int32, (batch, max_time), 1)
    onehot = (t_iota == ti_ref[...]).astype(jnp.float32)
    tm = jnp.dot(onehot, wtime_ref[...], preferred_element_type=jnp.float32)
    eb_ref[...] = jnp.dot(tm, p1, preferred_element_type=jnp.float32)

    def body(i, _):
        b = lax.rem(i, NBUF)
        in_copy(i, b).wait()

        @pl.when(i >= NBUF)
        def _():
            out_copy(i - NBUF, b).wait()

        ef = ef_ref[pl.ds(i * CH, CH), :]
        out_buf[b] = (in_buf[b] + ef[:, None, :] + eb_ref[...][None, :, :])
        out_copy(i, b).start()

        @pl.when(i + NBUF < n_chunks)
        def _():
            in_copy(i + NBUF, b).start()

        return 0

    lax.fori_loop(0, n_chunks, body, 0)

    for b in range(NBUF):
        out_copy(0, b).wait()


@jax.jit
def kernel(x, time_indices, W_ft, W_time, W_tk):
    num_features, batch, d_model = x.shape
    ti = time_indices.astype(jnp.int32).reshape(batch, 1)
    return pl.pallas_call(
        _pipeline_kernel,
        in_specs=[
            pl.BlockSpec(memory_space=pltpu.MemorySpace.VMEM),
            pl.BlockSpec(memory_space=pltpu.MemorySpace.VMEM),
            pl.BlockSpec(memory_space=pltpu.MemorySpace.VMEM),
            pl.BlockSpec(memory_space=pltpu.MemorySpace.VMEM),
            pl.BlockSpec(memory_space=pl.ANY),
        ],
        out_specs=pl.BlockSpec(memory_space=pl.ANY),
        out_shape=jax.ShapeDtypeStruct(x.shape, x.dtype),
        scratch_shapes=[
            pltpu.VMEM((NBUF, CH, batch, d_model), jnp.float32),
            pltpu.VMEM((NBUF, CH, batch, d_model), jnp.float32),
            pltpu.VMEM((num_features, d_model), jnp.float32),
            pltpu.VMEM((batch, d_model), jnp.float32),
            pltpu.SemaphoreType.DMA((NBUF,)),
            pltpu.SemaphoreType.DMA((NBUF,)),
        ],
    )(ti, W_ft, W_time, W_tk, x)
